# baseline (speedup 1.0000x reference)
import functools

import jax
import jax.numpy as jnp
from jax import lax
from jax.experimental import pallas as pl
from jax.experimental.pallas import tpu as pltpu
from jax.experimental.pallas import tpu_sc as plsc

N = 10000
M = 32
ORIG = 92
NBR = 16
AF = 64
HF = 128
NCONV = 3
NCRY = 100
PER = 100
E = N * M

NW = 32
GB = 128
KCH = 80
NBUF = 4
QUADS = KCH // NBUF
E_PAD = NW * KCH * GB

AT = 200
ET = AT * M
GRID_E = N // AT

_EPS = 1e-5
_FE = float(E)
_FN = float(N)


def _softplus(x):
    return jnp.maximum(x, 0.0) + jnp.log1p(jnp.exp(-jnp.abs(x)))


def _make_sc_gather():
    mesh = plsc.VectorSubcoreMesh(
        core_axis_name="c", subcore_axis_name="s", num_cores=2, num_subcores=16
    )

    @functools.partial(
        pl.kernel,
        out_type=jax.ShapeDtypeStruct((E_PAD, AF), jnp.float32),
        mesh=mesh,
        scratch_types=[
            pltpu.VMEM((KCH, GB), jnp.int32),
            pltpu.VMEM((NBUF, GB, AF), jnp.float32),
            pltpu.SemaphoreType.DMA,
            pltpu.SemaphoreType.DMA,
            pltpu.SemaphoreType.DMA,
            pltpu.SemaphoreType.DMA,
        ],
        compiler_params=pltpu.CompilerParams(use_tc_tiling_on_sc=False),
    )
    def gather_k(tbl_hbm, idx_hbm, out_hbm, idx_v, rows_v, s0, s1, s2, s3):
        sems = (s0, s1, s2, s3)
        wid = lax.axis_index("s") * 2 + lax.axis_index("c")
        base = wid * (KCH * GB)
        pltpu.sync_copy(idx_hbm.at[wid], idx_v)

        for b in range(NBUF):
            pltpu.async_copy(tbl_hbm.at[idx_v.at[b]], rows_v.at[b], sems[b])

        def body(q, carry):
            j = q * NBUF
            for b in range(NBUF):
                pltpu.make_async_copy(
                    tbl_hbm.at[idx_v.at[0]], rows_v.at[b], sems[b]
                ).wait()
                pltpu.sync_copy(
                    rows_v.at[b], out_hbm.at[pl.ds(base + (j + b) * GB, GB)]
                )
                pltpu.async_copy(
                    tbl_hbm.at[idx_v.at[j + b + NBUF]], rows_v.at[b], sems[b]
                )
            return carry

        lax.fori_loop(0, QUADS - 1, body, 0)

        jlast = (QUADS - 1) * NBUF
        for b in range(NBUF):
            pltpu.make_async_copy(
                tbl_hbm.at[idx_v.at[0]], rows_v.at[b], sems[b]
            ).wait()
            pltpu.sync_copy(
                rows_v.at[b], out_hbm.at[pl.ds(base + (jlast + b) * GB, GB)]
            )

    return gather_k


_SC_GATHER_CACHE = []


def _sc_gather(tbl, idx3):
    if not _SC_GATHER_CACHE:
        _SC_GATHER_CACHE.append(_make_sc_gather())
    return _SC_GATHER_CACHE[0](tbl, idx3)


def _embed_body(af_ref, ew_ref, eb_ref, ws_ref, cb_ref, x_ref, p_ref):
    x = jnp.dot(af_ref[...], ew_ref[...], preferred_element_type=jnp.float32)
    x = x + eb_ref[...]
    x_ref[...] = x
    p_ref[...] = jnp.dot(x, ws_ref[...], preferred_element_type=jnp.float32) + cb_ref[...]


def _embed(atom_fea, emb_W, emb_b, w_self, conv_b):
    bt = 1000
    return pl.pallas_call(
        _embed_body,
        grid=(N // bt,),
        in_specs=[
            pl.BlockSpec((bt, ORIG), lambda i: (i, 0)),
            pl.BlockSpec((ORIG, AF), lambda i: (0, 0)),
            pl.BlockSpec((1, AF), lambda i: (0, 0)),
            pl.BlockSpec((AF, 2 * AF), lambda i: (0, 0)),
            pl.BlockSpec((1, 2 * AF), lambda i: (0, 0)),
        ],
        out_specs=[
            pl.BlockSpec((bt, AF), lambda i: (i, 0)),
            pl.BlockSpec((bt, 2 * AF), lambda i: (i, 0)),
        ],
        out_shape=[
            jax.ShapeDtypeStruct((N, AF), jnp.float32),
            jax.ShapeDtypeStruct((N, 2 * AF), jnp.float32),
        ],
    )(atom_fea, emb_W, emb_b.reshape(1, AF), w_self, conv_b.reshape(1, 2 * AF))


def _conv_body(gx_ref, nf_ref, p_ref, wn_ref, we_ref, g1_ref, b1_ref,
               ns_ref, s2_ref, q2_ref, ssum, ssq):
    i = pl.program_id(0)
    h = jnp.dot(gx_ref[...], wn_ref[...], preferred_element_type=jnp.float32)
    h = h + jnp.dot(
        nf_ref[...].reshape(ET, NBR), we_ref[...], preferred_element_type=jnp.float32
    )

    @pl.when(i == 0)
    def _():
        ssum[...] = jnp.zeros_like(ssum)
        ssq[...] = jnp.zeros_like(ssq)

    @pl.when(i < GRID_E)
    def _():
        p = p_ref[...]
        g = h + jnp.broadcast_to(p[:, None, :], (AT, M, 2 * AF)).reshape(ET, 2 * AF)
        ssum[...] += jnp.sum(g, axis=0, keepdims=True)
        ssq[...] += jnp.sum(g * g, axis=0, keepdims=True)

    @pl.when(i == GRID_E)
    def _():
        mu = ssum[...] * (1.0 / _FE)
        var = ssq[...] * (1.0 / _FE) - mu * mu
        scale = g1_ref[...] / jnp.sqrt(var + _EPS)
        ssum[...] = scale
        ssq[...] = b1_ref[...] - mu * scale

    @pl.when(i >= GRID_E)
    def _():
        scale = ssum[...]
        shift = ssq[...]
        pc = p_ref[...] * scale + shift
        g = h * scale + jnp.broadcast_to(pc[:, None, :], (AT, M, 2 * AF)).reshape(ET, 2 * AF)
        filt = jax.nn.sigmoid(g[:, :AF])
        core = _softplus(g[:, AF:])
        ns = jnp.sum((filt * core).reshape(AT, M, AF), axis=1)
        ns_ref[...] = ns

        @pl.when(i == GRID_E)
        def _():
            s2_ref[...] = jnp.zeros_like(s2_ref)
            q2_ref[...] = jnp.zeros_like(q2_ref)

        s2_ref[...] += jnp.sum(ns, axis=0, keepdims=True)
        q2_ref[...] += jnp.sum(ns * ns, axis=0, keepdims=True)


def _conv(gx, nbr_fea, p, w_nbr, w_edge, bn1_g, bn1_b):
    def emap(i):
        return (jnp.where(i < GRID_E, i, i - GRID_E), 0)

    def emap3(i):
        return (jnp.where(i < GRID_E, i, i - GRID_E), 0, 0)

    def omap(i):
        return (jnp.where(i < GRID_E, 0, i - GRID_E), 0)

    return pl.pallas_call(
        _conv_body,
        grid=(2 * GRID_E,),
        in_specs=[
            pl.BlockSpec((ET, AF), emap),
            pl.BlockSpec((AT, M, NBR), emap3),
            pl.BlockSpec((AT, 2 * AF), emap),
            pl.BlockSpec((AF, 2 * AF), lambda i: (0, 0)),
            pl.BlockSpec((NBR, 2 * AF), lambda i: (0, 0)),
            pl.BlockSpec((1, 2 * AF), lambda i: (0, 0)),
            pl.BlockSpec((1, 2 * AF), lambda i: (0, 0)),
        ],
        out_specs=[
            pl.BlockSpec((AT, AF), omap),
            pl.BlockSpec((1, AF), lambda i: (0, 0)),
            pl.BlockSpec((1, AF), lambda i: (0, 0)),
        ],
        out_shape=[
            jax.ShapeDtypeStruct((N, AF), jnp.float32),
            jax.ShapeDtypeStruct((1, AF), jnp.float32),
            jax.ShapeDtypeStruct((1, AF), jnp.float32),
        ],
        scratch_shapes=[
            pltpu.VMEM((1, 2 * AF), jnp.float32),
            pltpu.VMEM((1, 2 * AF), jnp.float32),
        ],
    )(gx, nbr_fea, p, w_nbr, w_edge, bn1_g.reshape(1, 2 * AF), bn1_b.reshape(1, 2 * AF))


def _bn2_coefs(s2_ref, q2_ref, g2_ref, b2_ref):
    mu = s2_ref[...] * (1.0 / _FN)
    var = q2_ref[...] * (1.0 / _FN) - mu * mu
    scale = g2_ref[...] / jnp.sqrt(var + _EPS)
    shift = b2_ref[...] - mu * scale
    return scale, shift


def _update_body_p(x_ref, ns_ref, s2_ref, q2_ref, g2_ref, b2_ref, ws_ref, cb_ref,
                   xo_ref, p_ref):
    scale, shift = _bn2_coefs(s2_ref, q2_ref, g2_ref, b2_ref)
    xn = _softplus(x_ref[...] + ns_ref[...] * scale + shift)
    xo_ref[...] = xn
    p_ref[...] = jnp.dot(xn, ws_ref[...], preferred_element_type=jnp.float32) + cb_ref[...]


def _update_body(x_ref, ns_ref, s2_ref, q2_ref, g2_ref, b2_ref, xo_ref):
    scale, shift = _bn2_coefs(s2_ref, q2_ref, g2_ref, b2_ref)
    xo_ref[...] = _softplus(x_ref[...] + ns_ref[...] * scale + shift)


def _update(x, ns, s2, q2, bn2_g, bn2_b, w_self=None, conv_b=None):
    bt = 1000
    base_specs = [
        pl.BlockSpec((bt, AF), lambda i: (i, 0)),
        pl.BlockSpec((bt, AF), lambda i: (i, 0)),
        pl.BlockSpec((1, AF), lambda i: (0, 0)),
        pl.BlockSpec((1, AF), lambda i: (0, 0)),
        pl.BlockSpec((1, AF), lambda i: (0, 0)),
        pl.BlockSpec((1, AF), lambda i: (0, 0)),
    ]
    args = (x, ns, s2, q2, bn2_g.reshape(1, AF), bn2_b.reshape(1, AF))
    if w_self is None:
        return pl.pallas_call(
            _update_body,
            grid=(N // bt,),
            in_specs=base_specs,
            out_specs=pl.BlockSpec((bt, AF), lambda i: (i, 0)),
            out_shape=jax.ShapeDtypeStruct((N, AF), jnp.float32),
        )(*args)
    return pl.pallas_call(
        _update_body_p,
        grid=(N // bt,),
        in_specs=base_specs + [
            pl.BlockSpec((AF, 2 * AF), lambda i: (0, 0)),
            pl.BlockSpec((1, 2 * AF), lambda i: (0, 0)),
        ],
        out_specs=[
            pl.BlockSpec((bt, AF), lambda i: (i, 0)),
            pl.BlockSpec((bt, 2 * AF), lambda i: (i, 0)),
        ],
        out_shape=[
            jax.ShapeDtypeStruct((N, AF), jnp.float32),
            jax.ShapeDtypeStruct((N, 2 * AF), jnp.float32),
        ],
    )(*args, w_self, conv_b.reshape(1, 2 * AF))


def _head_body(x_ref, fw_ref, fb_ref, ow_ref, ob_ref, out_ref):
    crys = jnp.mean(x_ref[...].reshape(NCRY, PER, AF), axis=1)
    h = _softplus(
        jnp.dot(_softplus(crys), fw_ref[...], preferred_element_type=jnp.float32)
        + fb_ref[...]
    )
    out_ref[...] = jnp.dot(h, ow_ref[...], preferred_element_type=jnp.float32) + ob_ref[...]


def _head(x, fc_W, fc_b, out_W, out_b):
    return pl.pallas_call(
        _head_body,
        in_specs=[
            pl.BlockSpec((N, AF), lambda: (0, 0)),
            pl.BlockSpec((AF, HF), lambda: (0, 0)),
            pl.BlockSpec((1, HF), lambda: (0, 0)),
            pl.BlockSpec((HF, 1), lambda: (0, 0)),
            pl.BlockSpec((1, 1), lambda: (0, 0)),
        ],
        out_specs=pl.BlockSpec((NCRY, 1), lambda: (0, 0)),
        out_shape=jax.ShapeDtypeStruct((NCRY, 1), jnp.float32),
    )(x, fc_W, fc_b.reshape(1, HF), out_W, out_b.reshape(1, 1))


def kernel(atom_fea, nbr_fea, nbr_fea_idx, crystal_atom_idx, atom_type,
           nbr_type, nbr_dist, pair_type, global_fea, pool_atom_idx,
           emb_W, emb_b, convW, convb, bn1_g, bn1_b, bn2_g, bn2_b,
           fc_W, fc_b, out_W, out_b):
    flat_idx = nbr_fea_idx.astype(jnp.int32).reshape(-1)
    idx3 = jnp.concatenate(
        [flat_idx, jnp.zeros((E_PAD - E,), jnp.int32)]
    ).reshape(NW, KCH, GB)

    w_self = convW[:, :AF, :]
    w_nbr = convW[:, AF:2 * AF, :]
    w_edge = convW[:, 2 * AF:, :]

    x, p = _embed(atom_fea, emb_W, emb_b, w_self[0], convb[0])

    for i in range(NCONV):
        gx = _sc_gather(x, idx3)
        ns, s2, q2 = _conv(gx, nbr_fea, p, w_nbr[i], w_edge[i], bn1_g[i], bn1_b[i])
        if i + 1 < NCONV:
            x, p = _update(x, ns, s2, q2, bn2_g[i], bn2_b[i], w_self[i + 1], convb[i + 1])
        else:
            x = _update(x, ns, s2, q2, bn2_g[i], bn2_b[i])

    return _head(x, fc_W, fc_b, out_W, out_b)


# speedup vs baseline: 1.0695x; 1.0695x over previous
import functools

import jax
import jax.numpy as jnp
from jax import lax
from jax.experimental import pallas as pl
from jax.experimental.pallas import tpu as pltpu
from jax.experimental.pallas import tpu_sc as plsc

N = 10000
M = 32
ORIG = 92
NBR = 16
AF = 64
HF = 128
NCONV = 3
NCRY = 100
PER = 100
E = N * M

NW = 32
GB = 128
KCH = 80
NBUF = 4
QUADS = KCH // NBUF
E_PAD = NW * KCH * GB

AT = 200
ET = AT * M
GRID_E = N // AT

_EPS = 1e-5
_FE = float(E)
_FN = float(N)


def _softplus(x):
    return jnp.maximum(x, 0.0) + jnp.log1p(jnp.exp(-jnp.abs(x)))


def _make_sc_gather():
    mesh = plsc.VectorSubcoreMesh(
        core_axis_name="c", subcore_axis_name="s", num_cores=2, num_subcores=16
    )

    @functools.partial(
        pl.kernel,
        out_type=jax.ShapeDtypeStruct((E_PAD, AF), jnp.float32),
        mesh=mesh,
        scratch_types=[
            pltpu.VMEM((KCH, GB), jnp.int32),
            pltpu.VMEM((NBUF, GB, AF), jnp.float32),
            pltpu.SemaphoreType.DMA,
            pltpu.SemaphoreType.DMA,
            pltpu.SemaphoreType.DMA,
            pltpu.SemaphoreType.DMA,
        ],
        compiler_params=pltpu.CompilerParams(use_tc_tiling_on_sc=False),
    )
    def gather_k(tbl_hbm, idx_hbm, out_hbm, idx_v, rows_v, s0, s1, s2, s3):
        sems = (s0, s1, s2, s3)
        wid = lax.axis_index("s") * 2 + lax.axis_index("c")
        base = wid * (KCH * GB)
        pltpu.sync_copy(idx_hbm.at[wid], idx_v)

        for b in range(NBUF):
            pltpu.async_copy(tbl_hbm.at[idx_v.at[b]], rows_v.at[b], sems[b])

        def body(q, carry):
            j = q * NBUF
            for b in range(NBUF):
                pltpu.make_async_copy(
                    tbl_hbm.at[idx_v.at[0]], rows_v.at[b], sems[b]
                ).wait()
                pltpu.sync_copy(
                    rows_v.at[b], out_hbm.at[pl.ds(base + (j + b) * GB, GB)]
                )
                pltpu.async_copy(
                    tbl_hbm.at[idx_v.at[j + b + NBUF]], rows_v.at[b], sems[b]
                )
            return carry

        lax.fori_loop(0, QUADS - 1, body, 0)

        jlast = (QUADS - 1) * NBUF
        for b in range(NBUF):
            pltpu.make_async_copy(
                tbl_hbm.at[idx_v.at[0]], rows_v.at[b], sems[b]
            ).wait()
            pltpu.sync_copy(
                rows_v.at[b], out_hbm.at[pl.ds(base + (jlast + b) * GB, GB)]
            )

    return gather_k


_SC_GATHER_CACHE = []


def _sc_gather(tbl, idx3):
    if not _SC_GATHER_CACHE:
        _SC_GATHER_CACHE.append(_make_sc_gather())
    return _SC_GATHER_CACHE[0](tbl, idx3)


def _embed_body(af_ref, ew_ref, eb_ref, ws_ref, cb_ref, x_ref, p_ref):
    x = jnp.dot(af_ref[...], ew_ref[...], preferred_element_type=jnp.float32)
    x = x + eb_ref[...]
    x_ref[...] = x
    p_ref[...] = jnp.dot(x, ws_ref[...], preferred_element_type=jnp.float32) + cb_ref[...]


def _embed(atom_fea, emb_W, emb_b, w_self, conv_b):
    bt = 1000
    return pl.pallas_call(
        _embed_body,
        grid=(N // bt,),
        in_specs=[
            pl.BlockSpec((bt, ORIG), lambda i: (i, 0)),
            pl.BlockSpec((ORIG, AF), lambda i: (0, 0)),
            pl.BlockSpec((1, AF), lambda i: (0, 0)),
            pl.BlockSpec((AF, 2 * AF), lambda i: (0, 0)),
            pl.BlockSpec((1, 2 * AF), lambda i: (0, 0)),
        ],
        out_specs=[
            pl.BlockSpec((bt, AF), lambda i: (i, 0)),
            pl.BlockSpec((bt, 2 * AF), lambda i: (i, 0)),
        ],
        out_shape=[
            jax.ShapeDtypeStruct((N, AF), jnp.float32),
            jax.ShapeDtypeStruct((N, 2 * AF), jnp.float32),
        ],
    )(atom_fea, emb_W, emb_b.reshape(1, AF), w_self, conv_b.reshape(1, 2 * AF))


def _edge_h(gx_ref, nf_ref, wn_ref, we_ref):
    h = jnp.dot(gx_ref[...], wn_ref[...], preferred_element_type=jnp.float32)
    return h + jnp.dot(
        nf_ref[...].reshape(ET, NBR), we_ref[...], preferred_element_type=jnp.float32
    )


def _stats_body(gx_ref, nf_ref, p_ref, wn_ref, we_ref, sum_ref, sq_ref, m0_ref, m0s):
    i = pl.program_id(0)
    h = _edge_h(gx_ref, nf_ref, wn_ref, we_ref)
    p = p_ref[...]
    g = h + jnp.broadcast_to(p[:, None, :], (AT, M, 2 * AF)).reshape(ET, 2 * AF)

    @pl.when(i == 0)
    def _():
        m0 = jnp.sum(g, axis=0, keepdims=True) * (1.0 / ET)
        m0s[...] = m0
        m0_ref[...] = m0
        sum_ref[...] = jnp.zeros_like(sum_ref)
        sq_ref[...] = jnp.zeros_like(sq_ref)

    d = g - m0s[...]
    sum_ref[...] += jnp.sum(d, axis=0, keepdims=True)
    sq_ref[...] += jnp.sum(d * d, axis=0, keepdims=True)


def _stats(gx, nbr_fea, p, w_nbr, w_edge):
    return pl.pallas_call(
        _stats_body,
        grid=(GRID_E,),
        in_specs=[
            pl.BlockSpec((ET, AF), lambda i: (i, 0)),
            pl.BlockSpec((AT, M, NBR), lambda i: (i, 0, 0)),
            pl.BlockSpec((AT, 2 * AF), lambda i: (i, 0)),
            pl.BlockSpec((AF, 2 * AF), lambda i: (0, 0)),
            pl.BlockSpec((NBR, 2 * AF), lambda i: (0, 0)),
        ],
        out_specs=[
            pl.BlockSpec((1, 2 * AF), lambda i: (0, 0)),
            pl.BlockSpec((1, 2 * AF), lambda i: (0, 0)),
            pl.BlockSpec((1, 2 * AF), lambda i: (0, 0)),
        ],
        out_shape=[
            jax.ShapeDtypeStruct((1, 2 * AF), jnp.float32),
            jax.ShapeDtypeStruct((1, 2 * AF), jnp.float32),
            jax.ShapeDtypeStruct((1, 2 * AF), jnp.float32),
        ],
        scratch_shapes=[
            pltpu.VMEM((1, 2 * AF), jnp.float32),
        ],
    )(gx, nbr_fea, p, w_nbr, w_edge)


def _apply_body(gx_ref, nf_ref, p_ref, wn_ref, we_ref, s1_ref, q1_ref, m0_ref,
                g1_ref, b1_ref, ns_ref, s2_ref, q2_ref, m2_ref, coef, m2s):
    i = pl.program_id(0)

    @pl.when(i == 0)
    def _():
        dmu = s1_ref[...] * (1.0 / _FE)
        mu = m0_ref[...] + dmu
        var = q1_ref[...] * (1.0 / _FE) - dmu * dmu
        scale = g1_ref[...] / jnp.sqrt(var + _EPS)
        coef[0:1, :] = scale
        coef[1:2, :] = b1_ref[...] - mu * scale

    h = _edge_h(gx_ref, nf_ref, wn_ref, we_ref)
    scale = coef[0:1, :]
    shift = coef[1:2, :]
    pc = p_ref[...] * scale + shift
    g = h * scale + jnp.broadcast_to(pc[:, None, :], (AT, M, 2 * AF)).reshape(ET, 2 * AF)
    filt = jax.nn.sigmoid(g[:, :AF])
    core = _softplus(g[:, AF:])
    ns = jnp.sum((filt * core).reshape(AT, M, AF), axis=1)
    ns_ref[...] = ns

    @pl.when(i == 0)
    def _():
        m2 = jnp.sum(ns, axis=0, keepdims=True) * (1.0 / AT)
        m2s[...] = m2
        m2_ref[...] = m2
        s2_ref[...] = jnp.zeros_like(s2_ref)
        q2_ref[...] = jnp.zeros_like(q2_ref)

    d = ns - m2s[...]
    s2_ref[...] += jnp.sum(d, axis=0, keepdims=True)
    q2_ref[...] += jnp.sum(d * d, axis=0, keepdims=True)


def _apply(gx, nbr_fea, p, w_nbr, w_edge, s1, q1, m0, bn1_g, bn1_b):
    return pl.pallas_call(
        _apply_body,
        grid=(GRID_E,),
        in_specs=[
            pl.BlockSpec((ET, AF), lambda i: (i, 0)),
            pl.BlockSpec((AT, M, NBR), lambda i: (i, 0, 0)),
            pl.BlockSpec((AT, 2 * AF), lambda i: (i, 0)),
            pl.BlockSpec((AF, 2 * AF), lambda i: (0, 0)),
            pl.BlockSpec((NBR, 2 * AF), lambda i: (0, 0)),
            pl.BlockSpec((1, 2 * AF), lambda i: (0, 0)),
            pl.BlockSpec((1, 2 * AF), lambda i: (0, 0)),
            pl.BlockSpec((1, 2 * AF), lambda i: (0, 0)),
            pl.BlockSpec((1, 2 * AF), lambda i: (0, 0)),
            pl.BlockSpec((1, 2 * AF), lambda i: (0, 0)),
        ],
        out_specs=[
            pl.BlockSpec((AT, AF), lambda i: (i, 0)),
            pl.BlockSpec((1, AF), lambda i: (0, 0)),
            pl.BlockSpec((1, AF), lambda i: (0, 0)),
            pl.BlockSpec((1, AF), lambda i: (0, 0)),
        ],
        out_shape=[
            jax.ShapeDtypeStruct((N, AF), jnp.float32),
            jax.ShapeDtypeStruct((1, AF), jnp.float32),
            jax.ShapeDtypeStruct((1, AF), jnp.float32),
            jax.ShapeDtypeStruct((1, AF), jnp.float32),
        ],
        scratch_shapes=[
            pltpu.VMEM((2, 2 * AF), jnp.float32),
            pltpu.VMEM((1, AF), jnp.float32),
        ],
    )(gx, nbr_fea, p, w_nbr, w_edge, s1, q1, m0,
      bn1_g.reshape(1, 2 * AF), bn1_b.reshape(1, 2 * AF))


def _bn2_coefs(s2_ref, q2_ref, m2_ref, g2_ref, b2_ref):
    dmu = s2_ref[...] * (1.0 / _FN)
    mu = m2_ref[...] + dmu
    var = q2_ref[...] * (1.0 / _FN) - dmu * dmu
    scale = g2_ref[...] / jnp.sqrt(var + _EPS)
    shift = b2_ref[...] - mu * scale
    return scale, shift


def _update_body_p(x_ref, ns_ref, s2_ref, q2_ref, m2_ref, g2_ref, b2_ref,
                   ws_ref, cb_ref, xo_ref, p_ref):
    scale, shift = _bn2_coefs(s2_ref, q2_ref, m2_ref, g2_ref, b2_ref)
    xn = _softplus(x_ref[...] + ns_ref[...] * scale + shift)
    xo_ref[...] = xn
    p_ref[...] = jnp.dot(xn, ws_ref[...], preferred_element_type=jnp.float32) + cb_ref[...]


def _update_body(x_ref, ns_ref, s2_ref, q2_ref, m2_ref, g2_ref, b2_ref, xo_ref):
    scale, shift = _bn2_coefs(s2_ref, q2_ref, m2_ref, g2_ref, b2_ref)
    xo_ref[...] = _softplus(x_ref[...] + ns_ref[...] * scale + shift)


def _update(x, ns, s2, q2, m2, bn2_g, bn2_b, w_self=None, conv_b=None):
    bt = 1000
    base_specs = [
        pl.BlockSpec((bt, AF), lambda i: (i, 0)),
        pl.BlockSpec((bt, AF), lambda i: (i, 0)),
        pl.BlockSpec((1, AF), lambda i: (0, 0)),
        pl.BlockSpec((1, AF), lambda i: (0, 0)),
        pl.BlockSpec((1, AF), lambda i: (0, 0)),
        pl.BlockSpec((1, AF), lambda i: (0, 0)),
        pl.BlockSpec((1, AF), lambda i: (0, 0)),
    ]
    args = (x, ns, s2, q2, m2, bn2_g.reshape(1, AF), bn2_b.reshape(1, AF))
    if w_self is None:
        return pl.pallas_call(
            _update_body,
            grid=(N // bt,),
            in_specs=base_specs,
            out_specs=pl.BlockSpec((bt, AF), lambda i: (i, 0)),
            out_shape=jax.ShapeDtypeStruct((N, AF), jnp.float32),
        )(*args)
    return pl.pallas_call(
        _update_body_p,
        grid=(N // bt,),
        in_specs=base_specs + [
            pl.BlockSpec((AF, 2 * AF), lambda i: (0, 0)),
            pl.BlockSpec((1, 2 * AF), lambda i: (0, 0)),
        ],
        out_specs=[
            pl.BlockSpec((bt, AF), lambda i: (i, 0)),
            pl.BlockSpec((bt, 2 * AF), lambda i: (i, 0)),
        ],
        out_shape=[
            jax.ShapeDtypeStruct((N, AF), jnp.float32),
            jax.ShapeDtypeStruct((N, 2 * AF), jnp.float32),
        ],
    )(*args, w_self, conv_b.reshape(1, 2 * AF))


def _head_body(x_ref, fw_ref, fb_ref, ow_ref, ob_ref, out_ref):
    crys = jnp.mean(x_ref[...].reshape(NCRY, PER, AF), axis=1)
    h = _softplus(
        jnp.dot(_softplus(crys), fw_ref[...], preferred_element_type=jnp.float32)
        + fb_ref[...]
    )
    out_ref[...] = jnp.dot(h, ow_ref[...], preferred_element_type=jnp.float32) + ob_ref[...]


def _head(x, fc_W, fc_b, out_W, out_b):
    return pl.pallas_call(
        _head_body,
        in_specs=[
            pl.BlockSpec((N, AF), lambda: (0, 0)),
            pl.BlockSpec((AF, HF), lambda: (0, 0)),
            pl.BlockSpec((1, HF), lambda: (0, 0)),
            pl.BlockSpec((HF, 1), lambda: (0, 0)),
            pl.BlockSpec((1, 1), lambda: (0, 0)),
        ],
        out_specs=pl.BlockSpec((NCRY, 1), lambda: (0, 0)),
        out_shape=jax.ShapeDtypeStruct((NCRY, 1), jnp.float32),
    )(x, fc_W, fc_b.reshape(1, HF), out_W, out_b.reshape(1, 1))


def kernel(atom_fea, nbr_fea, nbr_fea_idx, crystal_atom_idx, atom_type,
           nbr_type, nbr_dist, pair_type, global_fea, pool_atom_idx,
           emb_W, emb_b, convW, convb, bn1_g, bn1_b, bn2_g, bn2_b,
           fc_W, fc_b, out_W, out_b):
    flat_idx = nbr_fea_idx.astype(jnp.int32).reshape(-1)
    idx3 = jnp.concatenate(
        [flat_idx, jnp.zeros((E_PAD - E,), jnp.int32)]
    ).reshape(NW, KCH, GB)

    w_self = convW[:, :AF, :]
    w_nbr = convW[:, AF:2 * AF, :]
    w_edge = convW[:, 2 * AF:, :]

    x, p = _embed(atom_fea, emb_W, emb_b, w_self[0], convb[0])

    for i in range(NCONV):
        gx = _sc_gather(x, idx3)
        s1, q1, m0 = _stats(gx, nbr_fea, p, w_nbr[i], w_edge[i])
        ns, s2, q2, m2 = _apply(gx, nbr_fea, p, w_nbr[i], w_edge[i], s1, q1, m0,
                                bn1_g[i], bn1_b[i])
        if i + 1 < NCONV:
            x, p = _update(x, ns, s2, q2, m2, bn2_g[i], bn2_b[i],
                           w_self[i + 1], convb[i + 1])
        else:
            x = _update(x, ns, s2, q2, m2, bn2_g[i], bn2_b[i])

    return _head(x, fc_W, fc_b, out_W, out_b)


# speedup vs baseline: 1.0938x; 1.0227x over previous
import functools

import jax
import jax.numpy as jnp
from jax import lax
from jax.experimental import pallas as pl
from jax.experimental.pallas import tpu as pltpu
from jax.experimental.pallas import tpu_sc as plsc

N = 10000
M = 32
ORIG = 92
NBR = 16
AF = 64
HF = 128
NCONV = 3
NCRY = 100
PER = 100
E = N * M

NW = 32
GB = 128
KCH = 80
NBUF = 4
QUADS = KCH // NBUF
E_PAD = NW * KCH * GB

AT = 200
ET = AT * M
GRID_E = N // AT

_EPS = 1e-5
_FE = float(E)
_FN = float(N)


def _softplus(x):
    return jnp.maximum(x, 0.0) + jnp.log1p(jnp.exp(-jnp.abs(x)))


def _make_sc_gather():
    mesh = plsc.VectorSubcoreMesh(
        core_axis_name="c", subcore_axis_name="s", num_cores=2, num_subcores=16
    )

    @functools.partial(
        pl.kernel,
        out_type=jax.ShapeDtypeStruct((E_PAD, AF), jnp.bfloat16),
        mesh=mesh,
        scratch_types=[
            pltpu.VMEM((KCH, GB), jnp.int32),
            pltpu.VMEM((GB, AF), jnp.bfloat16),
            pltpu.SemaphoreType.DMA,
        ],
        compiler_params=pltpu.CompilerParams(use_tc_tiling_on_sc=False),
    )
    def gather_k(tbl_hbm, idx_hbm, out_hbm, idx_v, rows_v, sem):
        wid = lax.axis_index("s") * 2 + lax.axis_index("c")
        base = wid * (KCH * GB)
        pltpu.sync_copy(idx_hbm.at[wid], idx_v)

        def body(j, carry):
            pltpu.async_copy(tbl_hbm.at[idx_v.at[j]], rows_v, sem).wait()
            pltpu.sync_copy(rows_v, out_hbm.at[pl.ds(base + j * GB, GB)])
            return carry

        lax.fori_loop(0, KCH, body, 0)

    return gather_k


_SC_GATHER_CACHE = []


def _sc_gather(tbl, idx3):
    if not _SC_GATHER_CACHE:
        _SC_GATHER_CACHE.append(_make_sc_gather())
    return _SC_GATHER_CACHE[0](tbl, idx3)


def _embed_body(af_ref, ew_ref, eb_ref, ws_ref, cb_ref, x_ref, xb_ref, p_ref):
    x = jnp.dot(af_ref[...], ew_ref[...], preferred_element_type=jnp.float32)
    x = x + eb_ref[...]
    x_ref[...] = x
    xb_ref[...] = x.astype(jnp.bfloat16)
    p_ref[...] = jnp.dot(x, ws_ref[...], preferred_element_type=jnp.float32) + cb_ref[...]


def _embed(atom_fea, emb_W, emb_b, w_self, conv_b):
    bt = 1000
    return pl.pallas_call(
        _embed_body,
        grid=(N // bt,),
        in_specs=[
            pl.BlockSpec((bt, ORIG), lambda i: (i, 0)),
            pl.BlockSpec((ORIG, AF), lambda i: (0, 0)),
            pl.BlockSpec((1, AF), lambda i: (0, 0)),
            pl.BlockSpec((AF, 2 * AF), lambda i: (0, 0)),
            pl.BlockSpec((1, 2 * AF), lambda i: (0, 0)),
        ],
        out_specs=[
            pl.BlockSpec((bt, AF), lambda i: (i, 0)),
            pl.BlockSpec((bt, AF), lambda i: (i, 0)),
            pl.BlockSpec((bt, 2 * AF), lambda i: (i, 0)),
        ],
        out_shape=[
            jax.ShapeDtypeStruct((N, AF), jnp.float32),
            jax.ShapeDtypeStruct((N, AF), jnp.bfloat16),
            jax.ShapeDtypeStruct((N, 2 * AF), jnp.float32),
        ],
    )(atom_fea, emb_W, emb_b.reshape(1, AF), w_self, conv_b.reshape(1, 2 * AF))


def _edge_h(gx_ref, nf_ref, wn_ref, we_ref):
    h = jnp.dot(gx_ref[...], wn_ref[...], preferred_element_type=jnp.float32)
    return h + jnp.dot(
        nf_ref[...].reshape(ET, NBR), we_ref[...], preferred_element_type=jnp.float32
    )


def _stats_body(gx_ref, nf_ref, p_ref, wn_ref, we_ref, sum_ref, sq_ref, m0_ref, m0s):
    i = pl.program_id(0)
    h = _edge_h(gx_ref, nf_ref, wn_ref, we_ref)
    p = p_ref[...]
    g = h + jnp.broadcast_to(p[:, None, :], (AT, M, 2 * AF)).reshape(ET, 2 * AF)

    @pl.when(i == 0)
    def _():
        m0 = jnp.sum(g, axis=0, keepdims=True) * (1.0 / ET)
        m0s[...] = m0
        m0_ref[...] = m0
        sum_ref[...] = jnp.zeros_like(sum_ref)
        sq_ref[...] = jnp.zeros_like(sq_ref)

    d = g - m0s[...]
    sum_ref[...] += jnp.sum(d, axis=0, keepdims=True)
    sq_ref[...] += jnp.sum(d * d, axis=0, keepdims=True)


def _stats(gx, nbr_fea, p, w_nbr, w_edge):
    return pl.pallas_call(
        _stats_body,
        grid=(GRID_E,),
        in_specs=[
            pl.BlockSpec((ET, AF), lambda i: (i, 0)),
            pl.BlockSpec((AT, M, NBR), lambda i: (i, 0, 0)),
            pl.BlockSpec((AT, 2 * AF), lambda i: (i, 0)),
            pl.BlockSpec((AF, 2 * AF), lambda i: (0, 0)),
            pl.BlockSpec((NBR, 2 * AF), lambda i: (0, 0)),
        ],
        out_specs=[
            pl.BlockSpec((1, 2 * AF), lambda i: (0, 0)),
            pl.BlockSpec((1, 2 * AF), lambda i: (0, 0)),
            pl.BlockSpec((1, 2 * AF), lambda i: (0, 0)),
        ],
        out_shape=[
            jax.ShapeDtypeStruct((1, 2 * AF), jnp.float32),
            jax.ShapeDtypeStruct((1, 2 * AF), jnp.float32),
            jax.ShapeDtypeStruct((1, 2 * AF), jnp.float32),
        ],
        scratch_shapes=[
            pltpu.VMEM((1, 2 * AF), jnp.float32),
        ],
    )(gx, nbr_fea, p, w_nbr, w_edge)


def _apply_body(gx_ref, nf_ref, p_ref, wn_ref, we_ref, s1_ref, q1_ref, m0_ref,
                g1_ref, b1_ref, ns_ref, s2_ref, q2_ref, m2_ref, coef, m2s):
    i = pl.program_id(0)

    @pl.when(i == 0)
    def _():
        dmu = s1_ref[...] * (1.0 / _FE)
        mu = m0_ref[...] + dmu
        var = q1_ref[...] * (1.0 / _FE) - dmu * dmu
        scale = g1_ref[...] / jnp.sqrt(var + _EPS)
        coef[0:1, :] = scale
        coef[1:2, :] = b1_ref[...] - mu * scale

    h = _edge_h(gx_ref, nf_ref, wn_ref, we_ref)
    scale = coef[0:1, :]
    shift = coef[1:2, :]
    pc = p_ref[...] * scale + shift
    g = h * scale + jnp.broadcast_to(pc[:, None, :], (AT, M, 2 * AF)).reshape(ET, 2 * AF)
    filt = jax.nn.sigmoid(g[:, :AF])
    core = _softplus(g[:, AF:])
    ns = jnp.sum((filt * core).reshape(AT, M, AF), axis=1)
    ns_ref[...] = ns

    @pl.when(i == 0)
    def _():
        m2 = jnp.sum(ns, axis=0, keepdims=True) * (1.0 / AT)
        m2s[...] = m2
        m2_ref[...] = m2
        s2_ref[...] = jnp.zeros_like(s2_ref)
        q2_ref[...] = jnp.zeros_like(q2_ref)

    d = ns - m2s[...]
    s2_ref[...] += jnp.sum(d, axis=0, keepdims=True)
    q2_ref[...] += jnp.sum(d * d, axis=0, keepdims=True)


def _apply(gx, nbr_fea, p, w_nbr, w_edge, s1, q1, m0, bn1_g, bn1_b):
    return pl.pallas_call(
        _apply_body,
        grid=(GRID_E,),
        in_specs=[
            pl.BlockSpec((ET, AF), lambda i: (i, 0)),
            pl.BlockSpec((AT, M, NBR), lambda i: (i, 0, 0)),
            pl.BlockSpec((AT, 2 * AF), lambda i: (i, 0)),
            pl.BlockSpec((AF, 2 * AF), lambda i: (0, 0)),
            pl.BlockSpec((NBR, 2 * AF), lambda i: (0, 0)),
            pl.BlockSpec((1, 2 * AF), lambda i: (0, 0)),
            pl.BlockSpec((1, 2 * AF), lambda i: (0, 0)),
            pl.BlockSpec((1, 2 * AF), lambda i: (0, 0)),
            pl.BlockSpec((1, 2 * AF), lambda i: (0, 0)),
            pl.BlockSpec((1, 2 * AF), lambda i: (0, 0)),
        ],
        out_specs=[
            pl.BlockSpec((AT, AF), lambda i: (i, 0)),
            pl.BlockSpec((1, AF), lambda i: (0, 0)),
            pl.BlockSpec((1, AF), lambda i: (0, 0)),
            pl.BlockSpec((1, AF), lambda i: (0, 0)),
        ],
        out_shape=[
            jax.ShapeDtypeStruct((N, AF), jnp.float32),
            jax.ShapeDtypeStruct((1, AF), jnp.float32),
            jax.ShapeDtypeStruct((1, AF), jnp.float32),
            jax.ShapeDtypeStruct((1, AF), jnp.float32),
        ],
        scratch_shapes=[
            pltpu.VMEM((2, 2 * AF), jnp.float32),
            pltpu.VMEM((1, AF), jnp.float32),
        ],
    )(gx, nbr_fea, p, w_nbr, w_edge, s1, q1, m0,
      bn1_g.reshape(1, 2 * AF), bn1_b.reshape(1, 2 * AF))


def _bn2_coefs(s2_ref, q2_ref, m2_ref, g2_ref, b2_ref):
    dmu = s2_ref[...] * (1.0 / _FN)
    mu = m2_ref[...] + dmu
    var = q2_ref[...] * (1.0 / _FN) - dmu * dmu
    scale = g2_ref[...] / jnp.sqrt(var + _EPS)
    shift = b2_ref[...] - mu * scale
    return scale, shift


def _update_body_p(x_ref, ns_ref, s2_ref, q2_ref, m2_ref, g2_ref, b2_ref,
                   ws_ref, cb_ref, xo_ref, xb_ref, p_ref):
    scale, shift = _bn2_coefs(s2_ref, q2_ref, m2_ref, g2_ref, b2_ref)
    xn = _softplus(x_ref[...] + ns_ref[...] * scale + shift)
    xo_ref[...] = xn
    xb_ref[...] = xn.astype(jnp.bfloat16)
    p_ref[...] = jnp.dot(xn, ws_ref[...], preferred_element_type=jnp.float32) + cb_ref[...]


def _update_body(x_ref, ns_ref, s2_ref, q2_ref, m2_ref, g2_ref, b2_ref, xo_ref):
    scale, shift = _bn2_coefs(s2_ref, q2_ref, m2_ref, g2_ref, b2_ref)
    xo_ref[...] = _softplus(x_ref[...] + ns_ref[...] * scale + shift)


def _update(x, ns, s2, q2, m2, bn2_g, bn2_b, w_self=None, conv_b=None):
    bt = 1000
    base_specs = [
        pl.BlockSpec((bt, AF), lambda i: (i, 0)),
        pl.BlockSpec((bt, AF), lambda i: (i, 0)),
        pl.BlockSpec((1, AF), lambda i: (0, 0)),
        pl.BlockSpec((1, AF), lambda i: (0, 0)),
        pl.BlockSpec((1, AF), lambda i: (0, 0)),
        pl.BlockSpec((1, AF), lambda i: (0, 0)),
        pl.BlockSpec((1, AF), lambda i: (0, 0)),
    ]
    args = (x, ns, s2, q2, m2, bn2_g.reshape(1, AF), bn2_b.reshape(1, AF))
    if w_self is None:
        return pl.pallas_call(
            _update_body,
            grid=(N // bt,),
            in_specs=base_specs,
            out_specs=pl.BlockSpec((bt, AF), lambda i: (i, 0)),
            out_shape=jax.ShapeDtypeStruct((N, AF), jnp.float32),
        )(*args)
    return pl.pallas_call(
        _update_body_p,
        grid=(N // bt,),
        in_specs=base_specs + [
            pl.BlockSpec((AF, 2 * AF), lambda i: (0, 0)),
            pl.BlockSpec((1, 2 * AF), lambda i: (0, 0)),
        ],
        out_specs=[
            pl.BlockSpec((bt, AF), lambda i: (i, 0)),
            pl.BlockSpec((bt, AF), lambda i: (i, 0)),
            pl.BlockSpec((bt, 2 * AF), lambda i: (i, 0)),
        ],
        out_shape=[
            jax.ShapeDtypeStruct((N, AF), jnp.float32),
            jax.ShapeDtypeStruct((N, AF), jnp.bfloat16),
            jax.ShapeDtypeStruct((N, 2 * AF), jnp.float32),
        ],
    )(*args, w_self, conv_b.reshape(1, 2 * AF))


def _head_body(x_ref, fw_ref, fb_ref, ow_ref, ob_ref, out_ref):
    crys = jnp.mean(x_ref[...].reshape(NCRY, PER, AF), axis=1)
    h = _softplus(
        jnp.dot(_softplus(crys), fw_ref[...], preferred_element_type=jnp.float32)
        + fb_ref[...]
    )
    out_ref[...] = jnp.dot(h, ow_ref[...], preferred_element_type=jnp.float32) + ob_ref[...]


def _head(x, fc_W, fc_b, out_W, out_b):
    return pl.pallas_call(
        _head_body,
        in_specs=[
            pl.BlockSpec((N, AF), lambda: (0, 0)),
            pl.BlockSpec((AF, HF), lambda: (0, 0)),
            pl.BlockSpec((1, HF), lambda: (0, 0)),
            pl.BlockSpec((HF, 1), lambda: (0, 0)),
            pl.BlockSpec((1, 1), lambda: (0, 0)),
        ],
        out_specs=pl.BlockSpec((NCRY, 1), lambda: (0, 0)),
        out_shape=jax.ShapeDtypeStruct((NCRY, 1), jnp.float32),
    )(x, fc_W, fc_b.reshape(1, HF), out_W, out_b.reshape(1, 1))


def kernel(atom_fea, nbr_fea, nbr_fea_idx, crystal_atom_idx, atom_type,
           nbr_type, nbr_dist, pair_type, global_fea, pool_atom_idx,
           emb_W, emb_b, convW, convb, bn1_g, bn1_b, bn2_g, bn2_b,
           fc_W, fc_b, out_W, out_b):
    flat_idx = nbr_fea_idx.astype(jnp.int32).reshape(-1)
    idx3 = jnp.concatenate(
        [flat_idx, jnp.zeros((E_PAD - E,), jnp.int32)]
    ).reshape(NW, KCH, GB)

    w_self = convW[:, :AF, :]
    w_nbr = convW[:, AF:2 * AF, :].astype(jnp.bfloat16)
    w_edge = convW[:, 2 * AF:, :].astype(jnp.bfloat16)
    nf_bf = nbr_fea.astype(jnp.bfloat16)

    x, xb, p = _embed(atom_fea, emb_W, emb_b, w_self[0], convb[0])

    for i in range(NCONV):
        gx = _sc_gather(xb, idx3)
        s1, q1, m0 = _stats(gx, nf_bf, p, w_nbr[i], w_edge[i])
        ns, s2, q2, m2 = _apply(gx, nf_bf, p, w_nbr[i], w_edge[i], s1, q1, m0,
                                bn1_g[i], bn1_b[i])
        if i + 1 < NCONV:
            x, xb, p = _update(x, ns, s2, q2, m2, bn2_g[i], bn2_b[i],
                               w_self[i + 1], convb[i + 1])
        else:
            x = _update(x, ns, s2, q2, m2, bn2_g[i], bn2_b[i])

    return _head(x, fc_W, fc_b, out_W, out_b)


# speedup vs baseline: 1.1196x; 1.0236x over previous
import functools

import jax
import jax.numpy as jnp
from jax import lax
from jax.experimental import pallas as pl
from jax.experimental.pallas import tpu as pltpu
from jax.experimental.pallas import tpu_sc as plsc

N = 10000
M = 32
ORIG = 92
NBR = 16
AF = 64
HF = 128
NCONV = 3
NCRY = 100
PER = 100
E = N * M
MH = M // 2

NW = 32
GB = 64
KCH = 80
E_PAD = NW * KCH * 2 * GB
EH = E // 2
EH_PAD = E_PAD // 2

AT = 200
ET = AT * M
ETH = ET // 2
GRID_E = N // AT

_EPS = 1e-5
_FE = float(E)
_FN = float(N)


def _softplus(x):
    return jnp.maximum(x, 0.0) + jnp.log1p(jnp.exp(-jnp.abs(x)))


def _make_sc_gather():
    mesh = plsc.VectorSubcoreMesh(
        core_axis_name="c", subcore_axis_name="s", num_cores=2, num_subcores=16
    )

    @functools.partial(
        pl.kernel,
        out_type=jax.ShapeDtypeStruct((EH_PAD, 2 * AF), jnp.float32),
        mesh=mesh,
        scratch_types=[
            pltpu.VMEM((2 * KCH, GB), jnp.int32),
            pltpu.VMEM((GB, AF), jnp.float32),
            pltpu.VMEM((GB, AF), jnp.float32),
            pltpu.SemaphoreType.DMA,
            pltpu.SemaphoreType.DMA,
        ],
        compiler_params=pltpu.CompilerParams(use_tc_tiling_on_sc=False),
    )
    def gather_k(tbl_hbm, idx_hbm, out_hbm, idx_v, rv_e, rv_o, sem_e, sem_o):
        wid = lax.axis_index("s") * 2 + lax.axis_index("c")
        base = wid * (KCH * GB)
        pltpu.sync_copy(idx_hbm.at[wid], idx_v)

        def body(j, carry):
            ce = pltpu.async_copy(tbl_hbm.at[idx_v.at[2 * j]], rv_e, sem_e)
            co = pltpu.async_copy(tbl_hbm.at[idx_v.at[2 * j + 1]], rv_o, sem_o)
            ce.wait()
            co.wait()
            r0 = base + j * GB
            pltpu.sync_copy(rv_e, out_hbm.at[pl.ds(r0, GB), pl.ds(0, AF)])
            pltpu.sync_copy(rv_o, out_hbm.at[pl.ds(r0, GB), pl.ds(AF, AF)])
            return carry

        lax.fori_loop(0, KCH, body, 0)

    return gather_k


_SC_GATHER_CACHE = []


def _sc_gather(tbl, idx3):
    if not _SC_GATHER_CACHE:
        _SC_GATHER_CACHE.append(_make_sc_gather())
    return _SC_GATHER_CACHE[0](tbl, idx3)


def _embed_body(af_ref, ew_ref, eb_ref, ws_ref, cb_ref, x_ref, p_ref):
    x = jnp.dot(af_ref[...], ew_ref[...], preferred_element_type=jnp.float32)
    x = x + eb_ref[...]
    x_ref[...] = x
    p_ref[...] = jnp.dot(x, ws_ref[...], preferred_element_type=jnp.float32) + cb_ref[...]


def _embed(atom_fea, emb_W, emb_b, w_self, conv_b):
    bt = 1000
    return pl.pallas_call(
        _embed_body,
        grid=(N // bt,),
        in_specs=[
            pl.BlockSpec((bt, ORIG), lambda i: (i, 0)),
            pl.BlockSpec((ORIG, AF), lambda i: (0, 0)),
            pl.BlockSpec((1, AF), lambda i: (0, 0)),
            pl.BlockSpec((AF, 2 * AF), lambda i: (0, 0)),
            pl.BlockSpec((1, 2 * AF), lambda i: (0, 0)),
        ],
        out_specs=[
            pl.BlockSpec((bt, AF), lambda i: (i, 0)),
            pl.BlockSpec((bt, 2 * AF), lambda i: (i, 0)),
        ],
        out_shape=[
            jax.ShapeDtypeStruct((N, AF), jnp.float32),
            jax.ShapeDtypeStruct((N, 2 * AF), jnp.float32),
        ],
    )(atom_fea, emb_W, emb_b.reshape(1, AF), w_self, conv_b.reshape(1, 2 * AF))


def _edge_pair_h(gx_ref, nf_ref, wn_ref, we_ref):
    gx = gx_ref[...]
    nf = nf_ref[...]
    he = jnp.dot(gx[:, :AF], wn_ref[...], preferred_element_type=jnp.float32)
    he = he + jnp.dot(nf[:, :NBR], we_ref[...], preferred_element_type=jnp.float32)
    ho = jnp.dot(gx[:, AF:], wn_ref[...], preferred_element_type=jnp.float32)
    ho = ho + jnp.dot(nf[:, NBR:], we_ref[...], preferred_element_type=jnp.float32)
    return he, ho


def _stats_body(gx_ref, nf_ref, p_ref, wn_ref, we_ref, sum_ref, sq_ref, m0_ref, m0s):
    i = pl.program_id(0)
    he, ho = _edge_pair_h(gx_ref, nf_ref, wn_ref, we_ref)
    p = p_ref[...]
    pb = jnp.broadcast_to(p[:, None, :], (AT, MH, 2 * AF)).reshape(ETH, 2 * AF)
    ge = he + pb
    go = ho + pb

    @pl.when(i == 0)
    def _():
        m0 = (jnp.sum(ge, axis=0, keepdims=True)
              + jnp.sum(go, axis=0, keepdims=True)) * (1.0 / ET)
        m0s[...] = m0
        m0_ref[...] = m0
        sum_ref[...] = jnp.zeros_like(sum_ref)
        sq_ref[...] = jnp.zeros_like(sq_ref)

    m0 = m0s[...]
    d = ge - m0
    e = go - m0
    sum_ref[...] += (jnp.sum(d, axis=0, keepdims=True)
                     + jnp.sum(e, axis=0, keepdims=True))
    sq_ref[...] += (jnp.sum(d * d, axis=0, keepdims=True)
                    + jnp.sum(e * e, axis=0, keepdims=True))


def _stats(gx, nf2, p, w_nbr, w_edge):
    return pl.pallas_call(
        _stats_body,
        grid=(GRID_E,),
        in_specs=[
            pl.BlockSpec((ETH, 2 * AF), lambda i: (i, 0)),
            pl.BlockSpec((ETH, 2 * NBR), lambda i: (i, 0)),
            pl.BlockSpec((AT, 2 * AF), lambda i: (i, 0)),
            pl.BlockSpec((AF, 2 * AF), lambda i: (0, 0)),
            pl.BlockSpec((NBR, 2 * AF), lambda i: (0, 0)),
        ],
        out_specs=[
            pl.BlockSpec((1, 2 * AF), lambda i: (0, 0)),
            pl.BlockSpec((1, 2 * AF), lambda i: (0, 0)),
            pl.BlockSpec((1, 2 * AF), lambda i: (0, 0)),
        ],
        out_shape=[
            jax.ShapeDtypeStruct((1, 2 * AF), jnp.float32),
            jax.ShapeDtypeStruct((1, 2 * AF), jnp.float32),
            jax.ShapeDtypeStruct((1, 2 * AF), jnp.float32),
        ],
        scratch_shapes=[
            pltpu.VMEM((1, 2 * AF), jnp.float32),
        ],
    )(gx, nf2, p, w_nbr, w_edge)


def _apply_body(gx_ref, nf_ref, p_ref, wn_ref, we_ref, s1_ref, q1_ref, m0_ref,
                g1_ref, b1_ref, ns_ref, s2_ref, q2_ref, m2_ref, coef, m2s):
    i = pl.program_id(0)

    @pl.when(i == 0)
    def _():
        dmu = s1_ref[...] * (1.0 / _FE)
        mu = m0_ref[...] + dmu
        var = q1_ref[...] * (1.0 / _FE) - dmu * dmu
        scale = g1_ref[...] / jnp.sqrt(var + _EPS)
        coef[0:1, :] = scale
        coef[1:2, :] = b1_ref[...] - mu * scale

    he, ho = _edge_pair_h(gx_ref, nf_ref, wn_ref, we_ref)
    scale = coef[0:1, :]
    shift = coef[1:2, :]
    pc = p_ref[...] * scale + shift
    pcb = jnp.broadcast_to(pc[:, None, :], (AT, MH, 2 * AF)).reshape(ETH, 2 * AF)
    qe = he * scale + pcb
    qo = ho * scale + pcb
    pe = jax.nn.sigmoid(qe[:, :AF]) * _softplus(qe[:, AF:])
    po = jax.nn.sigmoid(qo[:, :AF]) * _softplus(qo[:, AF:])
    ns = (jnp.sum(pe.reshape(AT, MH, AF), axis=1)
          + jnp.sum(po.reshape(AT, MH, AF), axis=1))
    ns_ref[...] = ns

    @pl.when(i == 0)
    def _():
        m2 = jnp.sum(ns, axis=0, keepdims=True) * (1.0 / AT)
        m2s[...] = m2
        m2_ref[...] = m2
        s2_ref[...] = jnp.zeros_like(s2_ref)
        q2_ref[...] = jnp.zeros_like(q2_ref)

    d = ns - m2s[...]
    s2_ref[...] += jnp.sum(d, axis=0, keepdims=True)
    q2_ref[...] += jnp.sum(d * d, axis=0, keepdims=True)


def _apply(gx, nf2, p, w_nbr, w_edge, s1, q1, m0, bn1_g, bn1_b):
    return pl.pallas_call(
        _apply_body,
        grid=(GRID_E,),
        in_specs=[
            pl.BlockSpec((ETH, 2 * AF), lambda i: (i, 0)),
            pl.BlockSpec((ETH, 2 * NBR), lambda i: (i, 0)),
            pl.BlockSpec((AT, 2 * AF), lambda i: (i, 0)),
            pl.BlockSpec((AF, 2 * AF), lambda i: (0, 0)),
            pl.BlockSpec((NBR, 2 * AF), lambda i: (0, 0)),
            pl.BlockSpec((1, 2 * AF), lambda i: (0, 0)),
            pl.BlockSpec((1, 2 * AF), lambda i: (0, 0)),
            pl.BlockSpec((1, 2 * AF), lambda i: (0, 0)),
            pl.BlockSpec((1, 2 * AF), lambda i: (0, 0)),
            pl.BlockSpec((1, 2 * AF), lambda i: (0, 0)),
        ],
        out_specs=[
            pl.BlockSpec((AT, AF), lambda i: (i, 0)),
            pl.BlockSpec((1, AF), lambda i: (0, 0)),
            pl.BlockSpec((1, AF), lambda i: (0, 0)),
            pl.BlockSpec((1, AF), lambda i: (0, 0)),
        ],
        out_shape=[
            jax.ShapeDtypeStruct((N, AF), jnp.float32),
            jax.ShapeDtypeStruct((1, AF), jnp.float32),
            jax.ShapeDtypeStruct((1, AF), jnp.float32),
            jax.ShapeDtypeStruct((1, AF), jnp.float32),
        ],
        scratch_shapes=[
            pltpu.VMEM((2, 2 * AF), jnp.float32),
            pltpu.VMEM((1, AF), jnp.float32),
        ],
    )(gx, nf2, p, w_nbr, w_edge, s1, q1, m0,
      bn1_g.reshape(1, 2 * AF), bn1_b.reshape(1, 2 * AF))


def _bn2_coefs(s2_ref, q2_ref, m2_ref, g2_ref, b2_ref):
    dmu = s2_ref[...] * (1.0 / _FN)
    mu = m2_ref[...] + dmu
    var = q2_ref[...] * (1.0 / _FN) - dmu * dmu
    scale = g2_ref[...] / jnp.sqrt(var + _EPS)
    shift = b2_ref[...] - mu * scale
    return scale, shift


def _update_body_p(x_ref, ns_ref, s2_ref, q2_ref, m2_ref, g2_ref, b2_ref,
                   ws_ref, cb_ref, xo_ref, p_ref):
    scale, shift = _bn2_coefs(s2_ref, q2_ref, m2_ref, g2_ref, b2_ref)
    xn = _softplus(x_ref[...] + ns_ref[...] * scale + shift)
    xo_ref[...] = xn
    p_ref[...] = jnp.dot(xn, ws_ref[...], preferred_element_type=jnp.float32) + cb_ref[...]


def _update_body(x_ref, ns_ref, s2_ref, q2_ref, m2_ref, g2_ref, b2_ref, xo_ref):
    scale, shift = _bn2_coefs(s2_ref, q2_ref, m2_ref, g2_ref, b2_ref)
    xo_ref[...] = _softplus(x_ref[...] + ns_ref[...] * scale + shift)


def _update(x, ns, s2, q2, m2, bn2_g, bn2_b, w_self=None, conv_b=None):
    bt = 1000
    base_specs = [
        pl.BlockSpec((bt, AF), lambda i: (i, 0)),
        pl.BlockSpec((bt, AF), lambda i: (i, 0)),
        pl.BlockSpec((1, AF), lambda i: (0, 0)),
        pl.BlockSpec((1, AF), lambda i: (0, 0)),
        pl.BlockSpec((1, AF), lambda i: (0, 0)),
        pl.BlockSpec((1, AF), lambda i: (0, 0)),
        pl.BlockSpec((1, AF), lambda i: (0, 0)),
    ]
    args = (x, ns, s2, q2, m2, bn2_g.reshape(1, AF), bn2_b.reshape(1, AF))
    if w_self is None:
        return pl.pallas_call(
            _update_body,
            grid=(N // bt,),
            in_specs=base_specs,
            out_specs=pl.BlockSpec((bt, AF), lambda i: (i, 0)),
            out_shape=jax.ShapeDtypeStruct((N, AF), jnp.float32),
        )(*args)
    return pl.pallas_call(
        _update_body_p,
        grid=(N // bt,),
        in_specs=base_specs + [
            pl.BlockSpec((AF, 2 * AF), lambda i: (0, 0)),
            pl.BlockSpec((1, 2 * AF), lambda i: (0, 0)),
        ],
        out_specs=[
            pl.BlockSpec((bt, AF), lambda i: (i, 0)),
            pl.BlockSpec((bt, 2 * AF), lambda i: (i, 0)),
        ],
        out_shape=[
            jax.ShapeDtypeStruct((N, AF), jnp.float32),
            jax.ShapeDtypeStruct((N, 2 * AF), jnp.float32),
        ],
    )(*args, w_self, conv_b.reshape(1, 2 * AF))


def _head_body(x_ref, fw_ref, fb_ref, ow_ref, ob_ref, out_ref):
    crys = jnp.mean(x_ref[...].reshape(NCRY, PER, AF), axis=1)
    h = _softplus(
        jnp.dot(_softplus(crys), fw_ref[...], preferred_element_type=jnp.float32)
        + fb_ref[...]
    )
    out_ref[...] = jnp.dot(h, ow_ref[...], preferred_element_type=jnp.float32) + ob_ref[...]


def _head(x, fc_W, fc_b, out_W, out_b):
    return pl.pallas_call(
        _head_body,
        in_specs=[
            pl.BlockSpec((N, AF), lambda: (0, 0)),
            pl.BlockSpec((AF, HF), lambda: (0, 0)),
            pl.BlockSpec((1, HF), lambda: (0, 0)),
            pl.BlockSpec((HF, 1), lambda: (0, 0)),
            pl.BlockSpec((1, 1), lambda: (0, 0)),
        ],
        out_specs=pl.BlockSpec((NCRY, 1), lambda: (0, 0)),
        out_shape=jax.ShapeDtypeStruct((NCRY, 1), jnp.float32),
    )(x, fc_W, fc_b.reshape(1, HF), out_W, out_b.reshape(1, 1))


def kernel(atom_fea, nbr_fea, nbr_fea_idx, crystal_atom_idx, atom_type,
           nbr_type, nbr_dist, pair_type, global_fea, pool_atom_idx,
           emb_W, emb_b, convW, convb, bn1_g, bn1_b, bn2_g, bn2_b,
           fc_W, fc_b, out_W, out_b):
    flat_idx = nbr_fea_idx.astype(jnp.int32).reshape(-1)
    fi = jnp.concatenate([flat_idx, jnp.zeros((E_PAD - E,), jnp.int32)])
    idx3 = fi.reshape(NW, KCH, GB, 2).transpose(0, 1, 3, 2).reshape(NW, 2 * KCH, GB)

    w_self = convW[:, :AF, :]
    w_nbr = convW[:, AF:2 * AF, :]
    w_edge = convW[:, 2 * AF:, :].astype(jnp.bfloat16)
    nf2 = nbr_fea.reshape(EH, 2 * NBR).astype(jnp.bfloat16)

    x, p = _embed(atom_fea, emb_W, emb_b, w_self[0], convb[0])

    for i in range(NCONV):
        gx = _sc_gather(x, idx3)
        s1, q1, m0 = _stats(gx, nf2, p, w_nbr[i], w_edge[i])
        ns, s2, q2, m2 = _apply(gx, nf2, p, w_nbr[i], w_edge[i], s1, q1, m0,
                                bn1_g[i], bn1_b[i])
        if i + 1 < NCONV:
            x, p = _update(x, ns, s2, q2, m2, bn2_g[i], bn2_b[i],
                           w_self[i + 1], convb[i + 1])
        else:
            x = _update(x, ns, s2, q2, m2, bn2_g[i], bn2_b[i])

    return _head(x, fc_W, fc_b, out_W, out_b)
